# Initial kernel scaffold; baseline (speedup 1.0000x reference)
import jax
import jax.numpy as jnp
from jax.experimental import pallas as pl


def kernel(Ca, mask, residue_idx, chain_labels, pe_W, pe_b, edge_W, ln_g, ln_b):
    raise NotImplementedError("write your pallas kernel here")


# speedup vs baseline: 6.6048x; 6.6048x over previous
import functools

import jax
import jax.numpy as jnp
from jax.experimental import pallas as pl

B, L, K = 2, 2048, 30
NUM_RBF = 16
NUM_PE = 16
EDGE_FEAT = 128
MAX_REL = 32
EDGE_IN = NUM_PE + NUM_RBF * 9 + 7

BL = 128
_BIG = 1e30
_HI = jax.lax.Precision.HIGHEST


def _nrm(x, eps=1e-12):
    n = jnp.linalg.norm(x, axis=-1, keepdims=True)
    return x / jnp.maximum(n, eps)


def _rbf_cols(d):
    step = (22.0 - 2.0) / (NUM_RBF - 1)
    mu = 2.0 + step * jax.lax.broadcasted_iota(
        jnp.int32, (1, NUM_RBF), 1).astype(jnp.float32)
    sigma = (22.0 - 2.0) / NUM_RBF
    z = (d - mu) / sigma
    return jnp.exp(-(z * z))


def _pairdist(a, bv):
    d = a - bv
    return jnp.sqrt(jnp.sum(d * d, axis=1, keepdims=True) + 1e-6)


def _edge_kernel(cat_ref, g_ref, gi_ref, mask_ref, maski_ref, pew_ref,
                 peb_ref, ew_ref, lng_ref, lnb_ref, e_ref, eidx_ref):
    cat = cat_ref[0]
    g = g_ref[0]
    mask_j = mask_ref[0]
    gi = gi_ref[0]
    mask_i = maski_ref[0]

    acc = jnp.zeros((BL, L), jnp.float32)
    for c in range(3):
        xi = gi[:, c].reshape(BL, 1)
        xj = cat[c, :].reshape(1, L)
        dx = xi - xj
        acc = acc + dx * dx
    m2 = mask_i * mask_j
    dmat = m2 * jnp.sqrt(acc + 1e-6)
    dmax = jnp.max(dmat, axis=1, keepdims=True)
    dadj = dmat + (1.0 - m2) * dmax

    iota = jax.lax.broadcasted_iota(jnp.int32, (BL, L), 1)
    dcur = dadj
    vals = []
    idxs = []
    for _ in range(K):
        mv = jnp.min(dcur, axis=1, keepdims=True)
        cand = jnp.where(dcur == mv, iota, L)
        ik = jnp.min(cand, axis=1, keepdims=True)
        vals.append(mv)
        idxs.append(ik)
        dcur = jnp.where(iota == ik, _BIG, dcur)

    eidx_ref[0] = jnp.concatenate(idxs, axis=1)

    cai, ca0i, ca2i = gi[:, 0:3], gi[:, 3:6], gi[:, 6:9]
    oi = gi[:, 9:18]
    ri = gi[:, 18:19]
    ci = gi[:, 19:20]

    pew = pew_ref[...]
    peb = peb_ref[...]
    ew = ew_ref[...]
    lng = lng_ref[...]
    lnb = lnb_ref[...]
    iota66 = jax.lax.broadcasted_iota(
        jnp.int32, (BL, 2 * MAX_REL + 2), 1)

    for k in range(K):
        ik = idxs[k]
        oh = (iota == ik).astype(jnp.float32)
        gj = jnp.dot(oh, g, precision=_HI)
        caj, ca0j, ca2j = gj[:, 0:3], gj[:, 3:6], gj[:, 6:9]
        oj = gj[:, 9:18]
        rj = gj[:, 18:19]
        cj = gj[:, 19:20]

        ch = (ci == cj).astype(jnp.float32)
        off = ri - rj
        d_pe = jnp.clip(off + MAX_REL, 0.0, 2.0 * MAX_REL) * ch \
            + (1.0 - ch) * (2.0 * MAX_REL + 1.0)
        oh_pe = (iota66 == d_pe.astype(jnp.int32)).astype(jnp.float32)
        e_pos = jnp.dot(oh_pe, pew, precision=_HI) + peb

        rbf = [
            _rbf_cols(vals[k]),
            _rbf_cols(_pairdist(ca0i, ca0j)),
            _rbf_cols(_pairdist(ca2i, ca2j)),
            _rbf_cols(_pairdist(ca0i, caj)),
            _rbf_cols(_pairdist(ca0i, ca2j)),
            _rbf_cols(_pairdist(cai, ca0j)),
            _rbf_cols(_pairdist(cai, ca2j)),
            _rbf_cols(_pairdist(ca2i, ca0j)),
            _rbf_cols(_pairdist(ca2i, caj)),
        ]

        dxv = caj - cai
        du = []
        for a in range(3):
            du.append(jnp.sum(oi[:, 3 * a:3 * a + 3] * dxv,
                              axis=1, keepdims=True))
        dun = jnp.sqrt(du[0] * du[0] + du[1] * du[1] + du[2] * du[2])
        dun = jnp.maximum(dun, 1e-12)
        du = [d / dun for d in du]

        r = {}
        for a in range(3):
            for bb in range(3):
                r[(a, bb)] = (
                    oi[:, a:a + 1] * oj[:, bb:bb + 1]
                    + oi[:, 3 + a:4 + a] * oj[:, 3 + bb:4 + bb]
                    + oi[:, 6 + a:7 + a] * oj[:, 6 + bb:7 + bb])
        rxx, ryy, rzz = r[(0, 0)], r[(1, 1)], r[(2, 2)]
        mag_x = 0.5 * jnp.sqrt(jnp.abs(1.0 + rxx - ryy - rzz) + 1e-8)
        mag_y = 0.5 * jnp.sqrt(jnp.abs(1.0 - rxx + ryy - rzz) + 1e-8)
        mag_z = 0.5 * jnp.sqrt(jnp.abs(1.0 - rxx - ryy + rzz) + 1e-8)
        qx = jnp.sign(r[(2, 1)] - r[(1, 2)]) * mag_x
        qy = jnp.sign(r[(0, 2)] - r[(2, 0)]) * mag_y
        qz = jnp.sign(r[(1, 0)] - r[(0, 1)]) * mag_z
        qw = jnp.sqrt(jax.nn.relu(1.0 + rxx + ryy + rzz) + 1e-8) / 2.0
        qn = jnp.sqrt(qx * qx + qy * qy + qz * qz + qw * qw)
        qn = jnp.maximum(qn, 1e-12)

        feat = jnp.concatenate(
            [e_pos] + rbf
            + [du[0], du[1], du[2], qx / qn, qy / qn, qz / qn, qw / qn],
            axis=1)

        ek = jnp.dot(feat, ew, precision=_HI)
        mu = jnp.mean(ek, axis=1, keepdims=True)
        xc = ek - mu
        var = jnp.mean(xc * xc, axis=1, keepdims=True)
        ek = xc / jnp.sqrt(var + 1e-5) * lng + lnb

        e_ref[0, :, k * EDGE_FEAT:(k + 1) * EDGE_FEAT] = ek


@functools.partial(jax.jit, static_argnames=())
def kernel(Ca, mask, residue_idx, chain_labels, pe_W, pe_b, edge_W,
           ln_g, ln_b):
    f32 = jnp.float32
    Ca = Ca.astype(f32)

    Ca0 = jnp.zeros_like(Ca).at[:, 1:, :].set(Ca[:, :-1, :])
    Ca2 = jnp.zeros_like(Ca).at[:, :-1, :].set(Ca[:, 1:, :])

    dX = Ca[:, 1:, :] - Ca[:, :-1, :]
    dn = jnp.linalg.norm(dX, axis=-1)
    mstep = ((dn > 3.6) & (dn < 4.0)).astype(f32)
    dX = dX * mstep[:, :, None]
    U = _nrm(dX)
    u_2 = U[:, :-2, :]
    u_1 = U[:, 1:-1, :]
    n_2 = _nrm(jnp.cross(u_2, u_1))
    o_1 = _nrm(u_2 - u_1)
    O = jnp.stack([o_1, n_2, jnp.cross(o_1, n_2)], 2)
    O = O.reshape(O.shape[0], O.shape[1], 9)
    O = jnp.pad(O, ((0, 0), (1, 2), (0, 0)))

    G = jnp.concatenate(
        [Ca, Ca0, Ca2, O,
         residue_idx.astype(f32)[..., None],
         chain_labels.astype(f32)[..., None]], axis=2)
    Ca_t = jnp.swapaxes(Ca, 1, 2)

    grid = (B, L // BL)
    E_flat, E_idx = pl.pallas_call(
        _edge_kernel,
        grid=grid,
        in_specs=[
            pl.BlockSpec((1, 3, L), lambda b, i: (b, 0, 0)),
            pl.BlockSpec((1, L, 20), lambda b, i: (b, 0, 0)),
            pl.BlockSpec((1, BL, 20), lambda b, i: (b, i, 0)),
            pl.BlockSpec((1, 1, L), lambda b, i: (b, 0, 0)),
            pl.BlockSpec((1, BL, 1), lambda b, i: (b, i, 0)),
            pl.BlockSpec((2 * MAX_REL + 2, NUM_PE), lambda b, i: (0, 0)),
            pl.BlockSpec((1, NUM_PE), lambda b, i: (0, 0)),
            pl.BlockSpec((EDGE_IN, EDGE_FEAT), lambda b, i: (0, 0)),
            pl.BlockSpec((1, EDGE_FEAT), lambda b, i: (0, 0)),
            pl.BlockSpec((1, EDGE_FEAT), lambda b, i: (0, 0)),
        ],
        out_specs=[
            pl.BlockSpec((1, BL, K * EDGE_FEAT), lambda b, i: (b, i, 0)),
            pl.BlockSpec((1, BL, K), lambda b, i: (b, i, 0)),
        ],
        out_shape=[
            jax.ShapeDtypeStruct((B, L, K * EDGE_FEAT), f32),
            jax.ShapeDtypeStruct((B, L, K), jnp.int32),
        ],
    )(Ca_t, G, G, mask.astype(f32).reshape(B, 1, L),
      mask.astype(f32).reshape(B, L, 1), pe_W.astype(f32),
      pe_b.astype(f32).reshape(1, NUM_PE), edge_W.astype(f32),
      ln_g.astype(f32).reshape(1, EDGE_FEAT),
      ln_b.astype(f32).reshape(1, EDGE_FEAT))

    return E_flat.reshape(B, L, K, EDGE_FEAT), E_idx
